# baseline (speedup 1.0000x reference)
import functools

import jax
import jax.numpy as jnp
from jax import lax
from jax.experimental import pallas as pl
from jax.experimental.pallas import tpu as pltpu
from jax.experimental.pallas import tpu_sc as plsc

_K = 16
_HALF = 256


def _gather_call(V, D, B):
    info = plsc.get_sparse_core_info()
    NC, NS = info.num_cores, info.num_subcores
    NW = NC * NS
    b_per_w = B // NW
    n_half = b_per_w // _HALF
    n_chunk = _HALF // _K
    mesh = plsc.VectorSubcoreMesh(core_axis_name="c", subcore_axis_name="s")

    @functools.partial(
        pl.kernel,
        mesh=mesh,
        out_type=jax.ShapeDtypeStruct((B, D), jnp.float32),
        scratch_types=[
            pltpu.VMEM((b_per_w,), jnp.int32),
            pltpu.VMEM((2, _K, D, 128), jnp.float32),
            pltpu.VMEM((_HALF, D), jnp.float32),
            pltpu.SemaphoreType.DMA,
            pltpu.SemaphoreType.DMA,
        ],
        compiler_params=pltpu.CompilerParams(needs_layout_passes=False),
    )
    def k(table_hbm, idx_hbm, out_hbm, idx_v, ring, out_v, sem0, sem1):
        wid = lax.axis_index("s") * NC + lax.axis_index("c")
        base = wid * b_per_w
        pltpu.sync_copy(idx_hbm.at[pl.ds(base, b_per_w)], idx_v)

        lane = lax.iota(jnp.int32, 16)
        neg_inf = jnp.int32(-2147483648)
        sems = [sem0, sem1]
        n_chunks_total = n_half * n_chunk

        def scalarize(ci):
            iv16 = idx_v[pl.ds(ci * _K, _K)]
            return [
                jnp.max(jnp.where(lane == j, iv16, neg_inf))
                for j in range(_K)
            ]

        def fire(ci, bank):
            rs = scalarize(ci)
            for j in range(_K):
                blk = pl.multiple_of(
                    jnp.bitwise_and(rs[j], jnp.int32(~127)), 128
                )
                pltpu.async_copy(
                    table_hbm.at[pl.ds(0, 8), pl.ds(blk, 128)],
                    ring.at[bank, j, pl.ds(0, 8)],
                    sems[bank],
                )
                pltpu.async_copy(
                    table_hbm.at[pl.ds(8, 8), pl.ds(blk, 128)],
                    ring.at[bank, j, pl.ds(8, 8)],
                    sems[bank],
                )

        def drain_extract(ci, lc, bank):
            rs = scalarize(ci)
            for j in range(_K):
                pltpu.make_async_copy(
                    table_hbm.at[:, pl.ds(0, 128)],
                    ring.at[bank, j],
                    sems[bank],
                ).wait()
            for j in range(_K):
                sub = jnp.bitwise_and(rs[j], jnp.int32(127))
                val = plsc.load_gather(
                    ring.at[bank, j], [lane, jnp.full((16,), sub, jnp.int32)]
                )
                row = lc * _K + j
                plsc.store_scatter(
                    out_v, [jnp.full((16,), row, jnp.int32), lane], val
                )

        fire(jnp.int32(0), 0)

        for half in range(n_half):
            def pair_body(p, _, half=half):
                lc0 = 2 * p
                ci0 = jnp.int32(half * n_chunk) + lc0
                fire(ci0 + 1, 1)
                drain_extract(ci0, lc0, 0)

                @pl.when(ci0 + 2 < n_chunks_total)
                def _():
                    fire(ci0 + 2, 0)

                drain_extract(ci0 + 1, lc0 + 1, 1)
                return 0

            lax.fori_loop(0, n_chunk // 2, pair_body, 0)
            pltpu.sync_copy(
                out_v, out_hbm.at[pl.ds(base + half * _HALF, _HALF)]
            )

    return k


def kernel(data, channelindex):
    V, D = data.shape
    (B,) = channelindex.shape
    return _gather_call(V, D, B)(data.T, channelindex.astype(jnp.int32))


# speedup vs baseline: 1.0302x; 1.0302x over previous
import functools

import jax
import jax.numpy as jnp
from jax import lax
from jax.experimental import pallas as pl
from jax.experimental.pallas import tpu as pltpu
from jax.experimental.pallas import tpu_sc as plsc

_K = 16
_HALF = 256


def _gather_call(V, D, B):
    info = plsc.get_sparse_core_info()
    NC, NS = info.num_cores, info.num_subcores
    NW = NC * NS
    b_per_w = B // NW
    n_half = b_per_w // _HALF
    n_chunk = _HALF // _K
    mesh = plsc.VectorSubcoreMesh(core_axis_name="c", subcore_axis_name="s")

    @functools.partial(
        pl.kernel,
        mesh=mesh,
        out_type=jax.ShapeDtypeStruct((B, D), jnp.float32),
        scratch_types=[
            pltpu.VMEM((b_per_w,), jnp.int32),
            pltpu.VMEM((2, _K, D, 128), jnp.float32),
            pltpu.VMEM((_HALF, D), jnp.float32),
            pltpu.SemaphoreType.DMA,
            pltpu.SemaphoreType.DMA,
        ],
        compiler_params=pltpu.CompilerParams(needs_layout_passes=False),
    )
    def k(table_hbm, idx_hbm, out_hbm, idx_v, ring, out_v, sem0, sem1):
        wid = lax.axis_index("s") * NC + lax.axis_index("c")
        base = wid * b_per_w
        pltpu.sync_copy(idx_hbm.at[pl.ds(base, b_per_w)], idx_v)

        lane = lax.iota(jnp.int32, 16)
        neg_inf = jnp.int32(-2147483648)
        sems = [sem0, sem1]
        n_chunks_total = n_half * n_chunk

        def scalar_at(ci, j):
            iv16 = idx_v[pl.ds(ci * _K, _K)]
            return jnp.max(jnp.where(lane == j, iv16, neg_inf))

        def fire(ci, bank):
            def body(j, _):
                r = scalar_at(ci, j)
                blk = pl.multiple_of(
                    jnp.bitwise_and(r, jnp.int32(~127)), 128
                )
                pltpu.async_copy(
                    table_hbm.at[:, pl.ds(blk, 128)],
                    ring.at[bank, j],
                    sems[bank],
                )
                return 0

            lax.fori_loop(0, _K, body, 0)

        def drain_extract(ci, lc, bank):
            def wait_body(j, _):
                pltpu.make_async_copy(
                    table_hbm.at[:, pl.ds(0, 128)],
                    ring.at[bank, j],
                    sems[bank],
                ).wait()
                return 0

            lax.fori_loop(0, _K, wait_body, 0)

            def ext_body(j, _):
                r = scalar_at(ci, j)
                sub = jnp.bitwise_and(r, jnp.int32(127))
                val = plsc.load_gather(
                    ring.at[bank, j], [lane, jnp.full((16,), sub, jnp.int32)]
                )
                row = lc * _K + j
                plsc.store_scatter(
                    out_v, [jnp.full((16,), row, jnp.int32), lane], val
                )
                return 0

            lax.fori_loop(0, _K, ext_body, 0)

        fire(jnp.int32(0), 0)

        for half in range(n_half):
            def pair_body(p, _, half=half):
                lc0 = 2 * p
                ci0 = jnp.int32(half * n_chunk) + lc0
                fire(ci0 + 1, 1)
                drain_extract(ci0, lc0, 0)

                @pl.when(ci0 + 2 < n_chunks_total)
                def _():
                    fire(ci0 + 2, 0)

                drain_extract(ci0 + 1, lc0 + 1, 1)
                return 0

            lax.fori_loop(0, n_chunk // 2, pair_body, 0)
            pltpu.sync_copy(
                out_v, out_hbm.at[pl.ds(base + half * _HALF, _HALF)]
            )

    return k


def kernel(data, channelindex):
    V, D = data.shape
    (B,) = channelindex.shape
    return _gather_call(V, D, B)(data.T, channelindex.astype(jnp.int32))
